# Initial kernel scaffold; baseline (speedup 1.0000x reference)
import jax
import jax.numpy as jnp
from jax.experimental import pallas as pl


def kernel(x, edge_index, batch, W1, b1, W2, b2, fc_w, fc_b):
    raise NotImplementedError("write your pallas kernel here")


# speedup vs baseline: 12.6748x; 12.6748x over previous
import functools

import jax
import jax.numpy as jnp
from jax import lax
from jax.experimental import pallas as pl
from jax.experimental.pallas import tpu as pltpu
from jax.experimental.pallas import tpu_sc as plsc

NC = 2
NS = 16
NW = NC * NS
LANE = 128
NUM_GRAPHS = 64


def _mesh():
    return plsc.VectorSubcoreMesh(core_axis_name="c", subcore_axis_name="s")


@functools.lru_cache(maxsize=None)
def _make_deg(npad, nblk, stripe):
    @functools.partial(
        pl.kernel,
        mesh=_mesh(),
        out_type=[
            jax.ShapeDtypeStruct((npad,), jnp.float32),
            jax.ShapeDtypeStruct((npad,), jnp.float32),
        ],
        scratch_types=[
            pltpu.VMEM((nblk, LANE), jnp.int32),
            pltpu.VMEM((LANE,), jnp.float32),
            pltpu.VMEM_SHARED((npad,), jnp.float32),
        ],
    )
    def deg_kernel(dst_hbm, zeros_hbm, out0, out1, idx_v, ones_v, acc_sh):
        c = lax.axis_index("c")
        s = lax.axis_index("s")
        wid = s * NC + c
        seg = pl.ds(s * stripe, stripe)
        pltpu.sync_copy(zeros_hbm, acc_sh.at[seg])
        pltpu.sync_copy(dst_hbm.at[wid], idx_v)
        for i in range(LANE // 16):
            ones_v[pl.ds(i * 16, 16)] = jnp.full((16,), 1.0, jnp.float32)
        plsc.subcore_barrier()

        def body(j, carry):
            pltpu.sync_copy(ones_v, acc_sh.at[idx_v.at[j]], add=True)
            return carry

        lax.fori_loop(0, nblk, body, 0)
        plsc.subcore_barrier()

        @pl.when(c == 0)
        def _():
            pltpu.sync_copy(acc_sh.at[seg], out0.at[seg])

        @pl.when(c == 1)
        def _():
            pltpu.sync_copy(acc_sh.at[seg], out1.at[seg])

    return deg_kernel


@functools.lru_cache(maxsize=None)
def _make_agg(npad, nblk, stripe, d):
    @functools.partial(
        pl.kernel,
        mesh=_mesh(),
        out_type=[
            jax.ShapeDtypeStruct((npad, d), jnp.float32),
            jax.ShapeDtypeStruct((npad, d), jnp.float32),
        ],
        scratch_types=[
            pltpu.VMEM((nblk, LANE), jnp.int32),
            pltpu.VMEM((nblk, LANE), jnp.int32),
            pltpu.VMEM((LANE, d), jnp.float32),
            pltpu.VMEM_SHARED((npad, d), jnp.float32),
            pltpu.SemaphoreType.DMA,
        ],
    )
    def agg_kernel(y_hbm, src_hbm, dst_hbm, zrows_hbm, out0, out1,
                   si_v, di_v, rows_v, acc_sh, sem):
        c = lax.axis_index("c")
        s = lax.axis_index("s")
        wid = s * NC + c
        seg = pl.ds(s * stripe, stripe)
        pltpu.sync_copy(zrows_hbm, acc_sh.at[seg])
        pltpu.sync_copy(src_hbm.at[wid], si_v)
        pltpu.sync_copy(dst_hbm.at[wid], di_v)
        plsc.subcore_barrier()

        def body(j, carry):
            pltpu.async_copy(y_hbm.at[si_v.at[j]], rows_v, sem).wait()
            pltpu.sync_copy(rows_v, acc_sh.at[di_v.at[j]], add=True)
            return carry

        lax.fori_loop(0, nblk, body, 0)
        plsc.subcore_barrier()

        @pl.when(c == 0)
        def _():
            pltpu.sync_copy(acc_sh.at[seg], out0.at[seg])

        @pl.when(c == 1)
        def _():
            pltpu.sync_copy(acc_sh.at[seg], out1.at[seg])

    return agg_kernel


def _prep_tc(x, w1, d0, d1):
    n = x.shape[0]
    d = w1.shape[1]

    def body(x_ref, w_ref, d0_ref, d1_ref, y_ref, dis_ref):
        deg = d0_ref[...][:n] + d1_ref[...][:n] + 1.0
        dis = lax.rsqrt(deg)
        dis_ref[...] = dis
        xw = jnp.dot(x_ref[...], w_ref[...], preferred_element_type=jnp.float32)
        y_ref[...] = xw * dis

    return pl.pallas_call(
        body,
        out_shape=[
            jax.ShapeDtypeStruct((n, d), jnp.float32),
            jax.ShapeDtypeStruct((n, 1), jnp.float32),
        ],
    )(x, w1, d0, d1)


def _mid_tc(p0, p1, y1, dis, b1, w2):
    n, d = y1.shape

    def body(p0_ref, p1_ref, y1_ref, dis_ref, b_ref, w_ref, y2_ref):
        acc = p0_ref[...][:n] + p1_ref[...][:n] + y1_ref[...]
        h = jnp.maximum(dis_ref[...] * acc + b_ref[...], 0.0)
        hw = jnp.dot(h, w_ref[...], preferred_element_type=jnp.float32)
        y2_ref[...] = hw * dis_ref[...]

    return pl.pallas_call(
        body,
        out_shape=jax.ShapeDtypeStruct((n, d), jnp.float32),
    )(p0, p1, y1, dis, b1, w2)


def _head_tc(q0, q1, y2, dis, b2, batch_row, fc_w, fc_b):
    n, d = y2.shape
    d_out = fc_w.shape[1]

    def body(q0_ref, q1_ref, y2_ref, dis_ref, b_ref, batch_ref, fw_ref,
             fb_ref, out_ref):
        acc = q0_ref[...][:n] + q1_ref[...][:n] + y2_ref[...]
        h = jnp.maximum(dis_ref[...] * acc + b_ref[...], 0.0)
        gids = lax.broadcasted_iota(jnp.int32, (NUM_GRAPHS, n), 0)
        m = (gids == batch_ref[...]).astype(jnp.float32)
        sums = jnp.dot(m, h, preferred_element_type=jnp.float32)
        counts = jnp.sum(m, axis=1, keepdims=True)
        pooled = sums / jnp.maximum(counts, 1.0)
        out_ref[...] = (
            jnp.dot(pooled, fw_ref[...], preferred_element_type=jnp.float32)
            + fb_ref[...]
        )

    return pl.pallas_call(
        body,
        out_shape=jax.ShapeDtypeStruct((NUM_GRAPHS, d_out), jnp.float32),
    )(q0, q1, y2, dis, b2, batch_row, fc_w, fc_b)


def kernel(x, edge_index, batch, W1, b1, W2, b2, fc_w, fc_b):
    n, _ = x.shape
    d = W1.shape[1]
    e = edge_index.shape[1]

    npad = (n // LANE + 2) * LANE
    stripe = npad // NS
    nblk = -(-e // (NW * LANE))
    pade = NW * nblk * LANE

    ei = edge_index.astype(jnp.int32)
    pad = pade - e
    src_p = jnp.concatenate(
        [ei[0], jnp.zeros((pad,), jnp.int32)]).reshape(NW, nblk, LANE)
    dst_p = jnp.concatenate(
        [ei[1], jnp.full((pad,), n, jnp.int32)]).reshape(NW, nblk, LANE)

    z1 = jnp.zeros((stripe,), jnp.float32)
    z2 = jnp.zeros((stripe, d), jnp.float32)

    deg0, deg1 = _make_deg(npad, nblk, stripe)(dst_p, z1)
    y1, dis = _prep_tc(x, W1, deg0.reshape(npad, 1), deg1.reshape(npad, 1))

    agg = _make_agg(npad, nblk, stripe, d)
    p0, p1 = agg(y1, src_p, dst_p, z2)
    y2 = _mid_tc(p0, p1, y1, dis, b1.reshape(1, d), W2)
    q0, q1 = agg(y2, src_p, dst_p, z2)

    out = _head_tc(q0, q1, y2, dis, b2.reshape(1, d),
                   batch.astype(jnp.int32).reshape(1, n),
                   fc_w, fc_b.reshape(1, fc_w.shape[1]))
    return out
